# baseline (speedup 1.0000x reference)
import jax
import jax.numpy as jnp
from jax.experimental import pallas as pl
from jax.experimental.pallas import tpu as pltpu

_K_CHEB = 3


def _round_up(v, m):
    return (v + m - 1) // m * m


def _mm_body(a_hi_ref, a_lo_ref, b_ref, o_ref):
    k = pl.program_id(2)

    @pl.when(k == 0)
    def _init():
        o_ref[...] = jnp.zeros_like(o_ref)

    b = b_ref[...]
    b_hi = b.astype(jnp.bfloat16)
    b_lo = (b - b_hi.astype(jnp.float32)).astype(jnp.bfloat16)
    a_hi = a_hi_ref[...]
    o_ref[...] += (
        jnp.dot(a_hi, b_hi, preferred_element_type=jnp.float32)
        + jnp.dot(a_lo_ref[...], b_hi, preferred_element_type=jnp.float32)
        + jnp.dot(a_hi, b_lo, preferred_element_type=jnp.float32))


def _matmul(a_hi, a_lo, b, bm=1024, bk=512):
    m, k = a_hi.shape
    _, n = b.shape
    bn = n
    assert m % bm == 0 and k % bk == 0, (a_hi.shape, b.shape, bm, bk)
    return pl.pallas_call(
        _mm_body,
        grid=(m // bm, n // bn, k // bk),
        in_specs=[
            pl.BlockSpec((bm, bk), lambda i, j, kk: (i, kk)),
            pl.BlockSpec((bm, bk), lambda i, j, kk: (i, kk)),
            pl.BlockSpec((bk, bn), lambda i, j, kk: (kk, j)),
        ],
        out_specs=pl.BlockSpec((bm, bn), lambda i, j, kk: (i, j)),
        out_shape=jax.ShapeDtypeStruct((m, n), jnp.float32),
        compiler_params=pltpu.CompilerParams(
            dimension_semantics=("parallel", "parallel", "arbitrary")),
    )(a_hi, a_lo, b)


def _build_dense_lap(edge_index, edge_weight, n, npad):
    row, col = edge_index[0], edge_index[1]
    w = jnp.where(row == col, 0.0, edge_weight)
    deg = jnp.zeros((n,), jnp.float32).at[row].add(w)
    deg_safe = jnp.where(deg > 0, deg, 1.0)
    dis = jnp.where(deg > 0, 1.0 / jnp.sqrt(deg_safe), 0.0)
    norm = -dis[row] * w * dis[col]
    A = jnp.zeros((npad, npad), jnp.float32).at[row, col].add(norm)
    return A


def _tconv_nm(X, p):
    npad, B, T, Ci = X.shape
    Ch = p['w1'].shape[0]
    rs = X.reshape(npad * B, T, Ci)
    dn = ('NWC', 'OIW', 'NWC')

    def cv(w, b):
        y = jax.lax.conv_general_dilated(
            rs, w[:, :, 0, :], (1,), 'VALID', dimension_numbers=dn,
            precision=jax.lax.Precision.HIGHEST)
        return y + b[None, None, :]

    P = cv(p['w1'], p['b1'])
    Q = jax.nn.sigmoid(cv(p['w2'], p['b2']))
    H = jax.nn.relu(P * Q + cv(p['w3'], p['b3']))
    return H.reshape(npad, B, T - 2, Ch)


def _cheb_nm(A2, X, Ws, bias):
    npad, B, T, C = X.shape
    F = B * T * C
    z = X.reshape(npad, F)
    Tx1 = _matmul(A2[0], A2[1], z)
    Tx2 = 2.0 * _matmul(A2[0], A2[1], Tx1) - z

    def wmul(t, W):
        return jnp.dot(t.reshape(npad * B * T, C), W.T,
                       precision=jax.lax.Precision.HIGHEST).reshape(npad, B, T, C)

    out = wmul(z, Ws[0]) + wmul(Tx1, Ws[1]) + wmul(Tx2, Ws[2]) + bias
    return out


def _bn_nm(X, gamma, beta, eps=1e-5):
    mean = jnp.mean(X, axis=(1, 2, 3), keepdims=True)
    var = jnp.var(X, axis=(1, 2, 3), keepdims=True)
    xh = (X - mean) / jnp.sqrt(var + eps)
    return xh * gamma[:, None, None, None] + beta[:, None, None, None]


def _stconv_nm(X, A2, p, npad, n):
    T0 = _tconv_nm(X, p['tc1'])
    z = _cheb_nm(A2, T0, p['cheb_w'], p['cheb_b'])
    T1 = _tconv_nm(jax.nn.relu(z), p['tc2'])
    g = jnp.pad(p['bn_g'], (0, npad - n), constant_values=1.0)
    b = jnp.pad(p['bn_b'], (0, npad - n))
    return _bn_nm(T1, g, b)


def _output_nm(X, p, npad, n):
    npad_, B, T, C = X.shape
    W1 = jnp.transpose(p['tc1_w'][:, :, :, 0], (2, 1, 0)).reshape(T * C, -1)
    y = jnp.dot(X.reshape(npad * B, T * C), W1,
                precision=jax.lax.Precision.HIGHEST) + p['tc1_b'][None, :]
    Co = y.shape[-1]
    y = y.reshape(npad, B, Co)
    yv = y[:n]
    mean = jnp.mean(yv, axis=(0, 2), keepdims=True)
    var = jnp.var(yv, axis=(0, 2), keepdims=True)
    ln_w = jnp.pad(p['ln_w'], ((0, npad - n), (0, 0)))
    ln_b = jnp.pad(p['ln_b'], ((0, npad - n), (0, 0)))
    y = (y - mean) / jnp.sqrt(var + eps_ln()) * ln_w[:, None, :] + ln_b[:, None, :]
    y = jnp.dot(y.reshape(npad * B, Co), p['tc2_w'][:, :, 0, 0].T,
                precision=jax.lax.Precision.HIGHEST) + p['tc2_b'][None, :]
    y = jnp.dot(y, p['fc_w'][:, :, 0, 0].T,
                precision=jax.lax.Precision.HIGHEST) + p['fc_b'][None, :]
    return y.reshape(npad, B)[:n].T.reshape(B, 1, 1, n)


def eps_ln():
    return 1e-5


def kernel(x, edge_index, edge_weight, params):
    B, T, n, C = x.shape
    npad = _round_up(n, 2048)
    A = _build_dense_lap(edge_index, edge_weight, n, npad)
    A_hi = A.astype(jnp.bfloat16)
    A_lo = (A - A_hi.astype(jnp.float32)).astype(jnp.bfloat16)
    A2 = (A_hi, A_lo)
    X = jnp.pad(jnp.transpose(x, (2, 0, 1, 3)), ((0, npad - n), (0, 0), (0, 0), (0, 0)))
    for l in range(2):
        X = _stconv_nm(X, A2, params['layer%d' % l], npad, n)
    return _output_nm(X, params['out'], npad, n)
